# baseline (speedup 1.0000x reference)
import functools

import jax
import jax.numpy as jnp
from jax import lax
from jax.experimental import pallas as pl
from jax.experimental.pallas import tpu as pltpu
from jax.experimental.pallas import tpu_sc as plsc

D_MODEL = 64
SCALE = 8.0

_NC = 2
_NS = 16
_NW = _NC * _NS
_CHUNK = 128
_LANES = 16
_DT = D_MODEL // 8
_GBUF = 4
_SBUF = 2
_SPAD = _CHUNK + 1

_AC = 384


@functools.lru_cache(maxsize=None)
def _make_transpose(vocab: int):
    mesh = plsc.VectorSubcoreMesh(core_axis_name="c", subcore_axis_name="s")
    n_full = vocab // _AC
    tail = vocab - n_full * _AC
    per_w = (n_full // _NW) & ~1
    n_extra = n_full - per_w * _NW
    assert per_w >= 4 and n_extra < 2 * _NW
    celems = _AC * D_MODEL

    @functools.partial(
        pl.kernel,
        mesh=mesh,
        out_type=jax.ShapeDtypeStruct((vocab * D_MODEL,), jnp.float32),
        compiler_params=pltpu.CompilerParams(
            use_tc_tiling_on_sc=True, needs_layout_passes=False),
        scratch_types=(
            [pltpu.VMEM((D_MODEL, _AC), jnp.float32)] * 2
            + [pltpu.VMEM((celems,), jnp.float32)] * 2
            + [pltpu.VMEM((D_MODEL, 128), jnp.float32)] * (1 if tail else 0)
            + [pltpu.SemaphoreType.DMA] * 4
        ),
    )
    def tk(src_hbm, out_hbm, *rest):
        ibuf = rest[0:2]
        obuf = rest[2:4]
        tbuf = rest[4] if tail else None
        sems = rest[5:] if tail else rest[4:]
        isem = sems[0:2]
        osem = sems[2:4]
        wid = lax.axis_index("s") * _NC + lax.axis_index("c")
        base = wid * per_w

        iota = lax.iota(jnp.int32, _LANES)
        dvecs = [(iota + j) & (_LANES - 1) for j in range(_LANES)]
        svecs = [iota * D_MODEL + dvecs[j] for j in range(_LANES)]

        def fire_in(i, b):
            off = pl.multiple_of(i * _AC, _AC)
            pltpu.async_copy(src_hbm.at[:, pl.ds(off, _AC)], ibuf[b], isem[b])

        def wait_in(b):
            pltpu.make_async_copy(
                src_hbm.at[:, pl.ds(0, _AC)], ibuf[b], isem[b]).wait()

        def fire_out(i, b):
            pltpu.async_copy(
                obuf[b], out_hbm.at[pl.ds(i * celems, celems)], osem[b])

        def wait_out(b):
            pltpu.make_async_copy(
                obuf[b], out_hbm.at[pl.ds(0, celems)], osem[b]).wait()

        def transpose(src, dst, ncols):
            def dgbody(dg, carry):
                d0 = dg * _LANES

                def gbody(g):
                    colv = iota + g * _LANES
                    sbase = g * (_LANES * D_MODEL) + d0
                    for j in range(_LANES):
                        v = plsc.load_gather(src, [dvecs[j] + d0, colv])
                        plsc.store_scatter(dst, [svecs[j] + sbase], v)

                plsc.parallel_loop(0, ncols // _LANES)(gbody)
                return carry

            lax.fori_loop(0, D_MODEL // _LANES, dgbody, 0)

        fire_in(base, 0)
        fire_in(base + 1, 1)
        for b in range(2):
            wait_in(b)
            transpose(ibuf[b], obuf[b], _AC)
            fire_out(base + b, b)
            fire_in(base + b + 2, b)

        def super_body(g, carry):
            k0 = 2 * g
            for b in range(2):
                wait_out(b)
                wait_in(b)
                transpose(ibuf[b], obuf[b], _AC)
                fire_out(base + k0 + b, b)
                fire_in(base + k0 + b + 2, b)
            return carry

        lax.fori_loop(1, per_w // 2 - 1, super_body, 0)

        k0 = per_w - 2
        for b in range(2):
            wait_out(b)
            wait_in(b)
            transpose(ibuf[b], obuf[b], _AC)
            fire_out(base + k0 + b, b)
        for b in range(2):
            wait_out(b)

        for r in range((n_extra + _NW - 1) // _NW):
            lo = r * _NW
            cnt = min(_NW, n_extra - lo)

            @pl.when(wid < cnt)
            def _(lo=lo):
                i = _NW * per_w + lo + wid
                fire_in(i, 0)
                wait_in(0)
                transpose(ibuf[0], obuf[0], _AC)
                fire_out(i, 0)
                wait_out(0)

        if tail:
            @pl.when(wid == _NW - 1)
            def _():
                toff = n_full * _AC
                doff = pl.multiple_of(toff + 0 * wid, 128)
                pltpu.async_copy(
                    src_hbm.at[:, pl.ds(doff, 128)], tbuf, isem[1])
                pltpu.make_async_copy(
                    src_hbm.at[:, pl.ds(0, 128)], tbuf, isem[1]).wait()
                transpose(tbuf, obuf[1], tail)
                n = tail * D_MODEL
                pltpu.async_copy(
                    obuf[1].at[pl.ds(0, n)],
                    out_hbm.at[pl.ds(toff * D_MODEL, n)], osem[1])
                pltpu.make_async_copy(
                    obuf[1].at[pl.ds(0, n)],
                    out_hbm.at[pl.ds(0, n)], osem[1]).wait()

    return tk


@functools.lru_cache(maxsize=None)
def _make_kernel(n_s: int, vocab: int):
    mesh = plsc.VectorSubcoreMesh(core_axis_name="c", subcore_axis_name="s")
    n_super = n_s // _GBUF
    assert n_s % _GBUF == 0 and n_super >= 2

    @functools.partial(
        pl.kernel,
        mesh=mesh,
        out_type=jax.ShapeDtypeStruct((n_s, _DT, _NW, 8, _CHUNK), jnp.float32),
        compiler_params=pltpu.CompilerParams(
            use_tc_tiling_on_sc=False, needs_layout_passes=False),
        scratch_types=(
            [pltpu.VMEM((n_s, _CHUNK), jnp.int32)]
            + [pltpu.VMEM((_CHUNK, D_MODEL), jnp.float32)] * _GBUF
            + [pltpu.VMEM((D_MODEL, _SPAD), jnp.float32)] * _SBUF
            + [pltpu.SemaphoreType.DMA] * (_GBUF + _SBUF)
        ),
    )
    def k(idx_hbm, table_hbm, out_hbm, idx_v, *rest):
        gbuf = rest[:_GBUF]
        sbuf = rest[_GBUF:_GBUF + _SBUF]
        gsem = rest[_GBUF + _SBUF:2 * _GBUF + _SBUF]
        ssem = rest[2 * _GBUF + _SBUF:]
        wid = lax.axis_index("s") * _NC + lax.axis_index("c")
        pltpu.sync_copy(idx_hbm.at[wid], idx_v)

        drows = [lax.iota(jnp.int32, _LANES) + (_LANES * dg)
                 for dg in range(D_MODEL // _LANES)]

        def fire_gather(s, b):
            pltpu.async_copy(table_hbm.at[idx_v.at[s]], gbuf[b], gsem[b])

        def wait_gather(b):
            pltpu.make_async_copy(
                table_hbm.at[idx_v.at[0]], gbuf[b], gsem[b]).wait()

        def fire_stores(s, t):
            for dt in range(_DT):
                pltpu.async_copy(
                    sbuf[t].at[pl.ds(dt * 8, 8), pl.ds(0, _CHUNK)],
                    out_hbm.at[s, dt, wid],
                    ssem[t],
                )

        def wait_stores(t):
            for dt in range(_DT):
                pltpu.make_async_copy(
                    sbuf[t].at[pl.ds(0, 8), pl.ds(0, _CHUNK)],
                    out_hbm.at[0, 0, wid],
                    ssem[t],
                ).wait()

        def transpose_scale(b, t):
            def blk(r0):
                for u in range(4):
                    r = r0 * 4 + u
                    col = jnp.full((_LANES,), 0, jnp.int32) + r
                    for dg in range(D_MODEL // _LANES):
                        v = gbuf[b][r, pl.ds(_LANES * dg, _LANES)]
                        plsc.store_scatter(sbuf[t], [drows[dg], col], v * SCALE)

            plsc.parallel_loop(0, _CHUNK // 4)(blk)

        for b in range(_GBUF):
            fire_gather(b, b)

        for b in range(_GBUF):
            t = b % _SBUF
            if b >= _SBUF:
                wait_stores(t)
            wait_gather(b)
            transpose_scale(b, t)
            fire_gather(b + _GBUF, b)
            fire_stores(b, t)

        def super_body(g, carry):
            s0 = g * _GBUF
            for b in range(_GBUF):
                t = b % _SBUF
                wait_stores(t)
                wait_gather(b)
                transpose_scale(b, t)
                fire_gather(s0 + b + _GBUF, b)
                fire_stores(s0 + b, t)
            return carry

        lax.fori_loop(1, n_super - 1, super_body, 0)

        s0 = (n_super - 1) * _GBUF
        for b in range(_GBUF):
            t = b % _SBUF
            wait_stores(t)
            wait_gather(b)
            transpose_scale(b, t)
            fire_stores(s0 + b, t)

        for t in range(_SBUF):
            wait_stores(t)

    return k


def kernel(x, lut):
    n_b, n_s = x.shape
    assert n_b == _NW * _CHUNK
    vocab, d = lut.shape
    idx = x.astype(jnp.int32).reshape(_NW, _CHUNK, n_s).transpose(0, 2, 1)
    flat = _make_transpose(vocab)(lut.T)
    raw = _make_kernel(n_s, vocab)(idx, flat.reshape(vocab, d))
    out = raw.transpose(2, 4, 0, 1, 3).reshape(n_b, n_s, D_MODEL)
    return out


# speedup vs baseline: 1.0022x; 1.0022x over previous
import functools

import jax
import jax.numpy as jnp
from jax import lax
from jax.experimental import pallas as pl
from jax.experimental.pallas import tpu as pltpu
from jax.experimental.pallas import tpu_sc as plsc

D_MODEL = 64
SCALE = 8.0

_NC = 2
_NS = 16
_NW = _NC * _NS
_CHUNK = 128
_LANES = 16
_DT = D_MODEL // 8
_GBUF = 4
_SBUF = 2
_SPAD = _CHUNK + 1

_AC = 256


@functools.lru_cache(maxsize=None)
def _make_transpose(vocab: int):
    mesh = plsc.VectorSubcoreMesh(core_axis_name="c", subcore_axis_name="s")
    n_full = vocab // _AC
    tail = vocab - n_full * _AC
    per_w = (n_full // _NW) & ~1
    n_extra = n_full - per_w * _NW
    assert per_w >= 4 and n_extra < 2 * _NW
    celems = _AC * D_MODEL

    @functools.partial(
        pl.kernel,
        mesh=mesh,
        out_type=jax.ShapeDtypeStruct((vocab * D_MODEL,), jnp.float32),
        compiler_params=pltpu.CompilerParams(
            use_tc_tiling_on_sc=True, needs_layout_passes=False),
        scratch_types=(
            [pltpu.VMEM((D_MODEL, _AC), jnp.float32)] * 2
            + [pltpu.VMEM((celems,), jnp.float32)] * 2
            + [pltpu.VMEM((D_MODEL, 128), jnp.float32)] * (1 if tail else 0)
            + [pltpu.SemaphoreType.DMA] * 4
        ),
    )
    def tk(src_hbm, out_hbm, *rest):
        ibuf = rest[0:2]
        obuf = rest[2:4]
        tbuf = rest[4] if tail else None
        sems = rest[5:] if tail else rest[4:]
        isem = sems[0:2]
        osem = sems[2:4]
        wid = lax.axis_index("s") * _NC + lax.axis_index("c")
        base = wid * per_w

        iota = lax.iota(jnp.int32, _LANES)
        dvecs = [(iota + j) & (_LANES - 1) for j in range(_LANES)]
        svecs = [iota * D_MODEL + dvecs[j] for j in range(_LANES)]

        def fire_in(i, b):
            off = pl.multiple_of(i * _AC, _AC)
            pltpu.async_copy(src_hbm.at[:, pl.ds(off, _AC)], ibuf[b], isem[b])

        def wait_in(b):
            pltpu.make_async_copy(
                src_hbm.at[:, pl.ds(0, _AC)], ibuf[b], isem[b]).wait()

        def fire_out(i, b):
            pltpu.async_copy(
                obuf[b], out_hbm.at[pl.ds(i * celems, celems)], osem[b])

        def wait_out(b):
            pltpu.make_async_copy(
                obuf[b], out_hbm.at[pl.ds(0, celems)], osem[b]).wait()

        def transpose(src, dst, ncols):
            def dgbody(dg, carry):
                d0 = dg * _LANES

                def gbody(g):
                    colv = iota + g * _LANES
                    sbase = g * (_LANES * D_MODEL) + d0
                    for j in range(_LANES):
                        v = plsc.load_gather(src, [dvecs[j] + d0, colv])
                        plsc.store_scatter(dst, [svecs[j] + sbase], v)

                plsc.parallel_loop(0, ncols // _LANES)(gbody)
                return carry

            lax.fori_loop(0, D_MODEL // _LANES, dgbody, 0)

        fire_in(base, 0)
        fire_in(base + 1, 1)
        for b in range(2):
            wait_in(b)
            transpose(ibuf[b], obuf[b], _AC)
            fire_out(base + b, b)
            fire_in(base + b + 2, b)

        def super_body(g, carry):
            k0 = 2 * g
            for b in range(2):
                wait_out(b)
                wait_in(b)
                transpose(ibuf[b], obuf[b], _AC)
                fire_out(base + k0 + b, b)
                fire_in(base + k0 + b + 2, b)
            return carry

        lax.fori_loop(1, per_w // 2 - 1, super_body, 0)

        k0 = per_w - 2
        for b in range(2):
            wait_out(b)
            wait_in(b)
            transpose(ibuf[b], obuf[b], _AC)
            fire_out(base + k0 + b, b)
        for b in range(2):
            wait_out(b)

        for r in range((n_extra + _NW - 1) // _NW):
            lo = r * _NW
            cnt = min(_NW, n_extra - lo)

            @pl.when(wid < cnt)
            def _(lo=lo):
                i = _NW * per_w + lo + wid
                fire_in(i, 0)
                wait_in(0)
                transpose(ibuf[0], obuf[0], _AC)
                fire_out(i, 0)
                wait_out(0)

        if tail:
            @pl.when(wid == _NW - 1)
            def _():
                toff = n_full * _AC
                doff = pl.multiple_of(toff + 0 * wid, 128)
                pltpu.async_copy(
                    src_hbm.at[:, pl.ds(doff, 128)], tbuf, isem[1])
                pltpu.make_async_copy(
                    src_hbm.at[:, pl.ds(0, 128)], tbuf, isem[1]).wait()
                transpose(tbuf, obuf[1], tail)
                n = tail * D_MODEL
                pltpu.async_copy(
                    obuf[1].at[pl.ds(0, n)],
                    out_hbm.at[pl.ds(toff * D_MODEL, n)], osem[1])
                pltpu.make_async_copy(
                    obuf[1].at[pl.ds(0, n)],
                    out_hbm.at[pl.ds(0, n)], osem[1]).wait()

    return tk


@functools.lru_cache(maxsize=None)
def _make_kernel(n_s: int, vocab: int):
    mesh = plsc.VectorSubcoreMesh(core_axis_name="c", subcore_axis_name="s")
    n_super = n_s // _GBUF
    assert n_s % _GBUF == 0 and n_super >= 2

    @functools.partial(
        pl.kernel,
        mesh=mesh,
        out_type=jax.ShapeDtypeStruct((n_s, _DT, _NW, 8, _CHUNK), jnp.float32),
        compiler_params=pltpu.CompilerParams(
            use_tc_tiling_on_sc=False, needs_layout_passes=False),
        scratch_types=(
            [pltpu.VMEM((n_s, _CHUNK), jnp.int32)]
            + [pltpu.VMEM((_CHUNK, D_MODEL), jnp.float32)] * _GBUF
            + [pltpu.VMEM((D_MODEL, _SPAD), jnp.float32)] * _SBUF
            + [pltpu.SemaphoreType.DMA] * (_GBUF + _SBUF)
        ),
    )
    def k(idx_hbm, table_hbm, out_hbm, idx_v, *rest):
        gbuf = rest[:_GBUF]
        sbuf = rest[_GBUF:_GBUF + _SBUF]
        gsem = rest[_GBUF + _SBUF:2 * _GBUF + _SBUF]
        ssem = rest[2 * _GBUF + _SBUF:]
        wid = lax.axis_index("s") * _NC + lax.axis_index("c")
        pltpu.sync_copy(idx_hbm.at[wid], idx_v)

        drows = [lax.iota(jnp.int32, _LANES) + (_LANES * dg)
                 for dg in range(D_MODEL // _LANES)]

        def fire_gather(s, b):
            pltpu.async_copy(table_hbm.at[idx_v.at[s]], gbuf[b], gsem[b])

        def wait_gather(b):
            pltpu.make_async_copy(
                table_hbm.at[idx_v.at[0]], gbuf[b], gsem[b]).wait()

        def fire_stores(s, t):
            for dt in range(_DT):
                pltpu.async_copy(
                    sbuf[t].at[pl.ds(dt * 8, 8), pl.ds(0, _CHUNK)],
                    out_hbm.at[s, dt, wid],
                    ssem[t],
                )

        def wait_stores(t):
            for dt in range(_DT):
                pltpu.make_async_copy(
                    sbuf[t].at[pl.ds(0, 8), pl.ds(0, _CHUNK)],
                    out_hbm.at[0, 0, wid],
                    ssem[t],
                ).wait()

        def transpose_scale(b, t):
            def blk(r0):
                for u in range(4):
                    r = r0 * 4 + u
                    col = jnp.full((_LANES,), 0, jnp.int32) + r
                    for dg in range(D_MODEL // _LANES):
                        v = gbuf[b][r, pl.ds(_LANES * dg, _LANES)]
                        plsc.store_scatter(sbuf[t], [drows[dg], col], v * SCALE)

            plsc.parallel_loop(0, _CHUNK // 4)(blk)

        for b in range(_GBUF):
            fire_gather(b, b)

        for b in range(_GBUF):
            t = b % _SBUF
            if b >= _SBUF:
                wait_stores(t)
            wait_gather(b)
            transpose_scale(b, t)
            fire_gather(b + _GBUF, b)
            fire_stores(b, t)

        def super_body(g, carry):
            s0 = g * _GBUF
            for b in range(_GBUF):
                t = b % _SBUF
                wait_stores(t)
                wait_gather(b)
                transpose_scale(b, t)
                fire_gather(s0 + b + _GBUF, b)
                fire_stores(s0 + b, t)
            return carry

        lax.fori_loop(1, n_super - 1, super_body, 0)

        s0 = (n_super - 1) * _GBUF
        for b in range(_GBUF):
            t = b % _SBUF
            wait_stores(t)
            wait_gather(b)
            transpose_scale(b, t)
            fire_stores(s0 + b, t)

        for t in range(_SBUF):
            wait_stores(t)

    return k


def kernel(x, lut):
    n_b, n_s = x.shape
    assert n_b == _NW * _CHUNK
    vocab, d = lut.shape
    idx = x.astype(jnp.int32).reshape(_NW, _CHUNK, n_s).transpose(0, 2, 1)
    flat = _make_transpose(vocab)(lut.T)
    raw = _make_kernel(n_s, vocab)(idx, flat.reshape(vocab, d))
    out = raw.transpose(2, 4, 0, 1, 3).reshape(n_b, n_s, D_MODEL)
    return out
